# baseline (speedup 1.0000x reference)
import functools

import jax
import jax.numpy as jnp
from jax import lax
from jax.experimental import pallas as pl
from jax.experimental.pallas import tpu as pltpu
from jax.experimental.pallas import tpu_sc as plsc

N = 10000
E = 320000
FEAT = 128
HID = 128

P = 10240
NW = 32
K = 128
EW_T = E // NW
NCH = -(-EW_T // K)
EP_T = NCH * K
RPT = P // NW

_mesh = plsc.VectorSubcoreMesh(core_axis_name="c", subcore_axis_name="s")


def _sigmoid(w):
    return 1.0 / (1.0 + jnp.exp(-w))


@functools.partial(
    pl.kernel,
    mesh=_mesh,
    out_type=jax.ShapeDtypeStruct((2 * P,), jnp.float32),
    scratch_types=[
        pltpu.VMEM((NCH, K), jnp.int32),
        pltpu.VMEM((NCH, K), jnp.float32),
        pltpu.VMEM((RPT,), jnp.float32),
        pltpu.VMEM_SHARED((P,), jnp.float32),
    ],
)
def _deg_kernel(colp, ewp, degp, col_v, val_v, zb, deg_sh):
    c = lax.axis_index("c")
    s = lax.axis_index("s")
    wid = c * 16 + s
    pltpu.sync_copy(colp.at[wid], col_v)
    pltpu.sync_copy(ewp.at[wid], val_v)

    def sig_body(j, carry):
        for f in range(8):
            sl = pl.ds(f * 16, 16)
            val_v[j, sl] = _sigmoid(val_v[j, sl])
        return carry

    lax.fori_loop(0, NCH, sig_body, None)

    def zb_body(t, carry):
        zb[pl.ds(t * 16, 16)] = jnp.zeros((16,), jnp.float32)
        return carry

    lax.fori_loop(0, RPT // 16, zb_body, None)
    pltpu.sync_copy(zb, deg_sh.at[pl.ds(s * RPT, RPT)])
    plsc.subcore_barrier()

    def sc_body(j, carry):
        pltpu.sync_copy(val_v.at[j], deg_sh.at[col_v.at[j]], add=True)
        return carry

    lax.fori_loop(0, NCH, sc_body, None)
    plsc.subcore_barrier()
    pltpu.sync_copy(deg_sh.at[pl.ds(s * RPT, RPT)], zb)
    pltpu.sync_copy(zb, degp.at[pl.ds(c * P + s * RPT, RPT)])


@functools.partial(
    pl.kernel,
    mesh=_mesh,
    out_type=jax.ShapeDtypeStruct((2 * P, HID), jnp.float32),
    scratch_types=[
        pltpu.VMEM((NCH, K), jnp.int32),
        pltpu.VMEM((NCH, K), jnp.int32),
        pltpu.VMEM((NCH, K), jnp.float32),
        pltpu.VMEM((K, HID), jnp.float32),
        pltpu.VMEM_SHARED((P, HID), jnp.float32),
        pltpu.SemaphoreType.DMA,
    ],
)
def _msg_kernel(rowp, colp, ewp, g, spart,
                row_v, col_v, scl_v, rows_buf, s_sh, gsem):
    c = lax.axis_index("c")
    s = lax.axis_index("s")
    wid = c * 16 + s
    pltpu.sync_copy(rowp.at[wid], row_v)
    pltpu.sync_copy(colp.at[wid], col_v)
    pltpu.sync_copy(ewp.at[wid], scl_v)

    def scl_body(j, carry):
        for f in range(8):
            sl = pl.ds(f * 16, 16)
            scl_v[j, sl] = _sigmoid(scl_v[j, sl])
        return carry

    lax.fori_loop(0, NCH, scl_body, None)

    def z_body(i, carry):
        for f in range(8):
            rows_buf[i, pl.ds(f * 16, 16)] = jnp.zeros((16,), jnp.float32)
        return carry

    lax.fori_loop(0, K, z_body, None)
    for t in range(RPT // K):
        pltpu.sync_copy(rows_buf, s_sh.at[pl.ds(s * RPT + t * K, K)])
    plsc.subcore_barrier()

    def chunk_body(j, carry):
        pltpu.async_copy(g.at[row_v.at[j]], rows_buf, gsem).wait()

        def row_scale(g, carry2):
            s16 = scl_v[j, pl.ds(g * 16, 16)]
            for l in range(16):
                spl = jnp.broadcast_to(s16[l], (16,))
                e = g * 16 + l
                for f in range(8):
                    sl = pl.ds(f * 16, 16)
                    rows_buf[e, sl] = rows_buf[e, sl] * spl
            return carry2

        lax.fori_loop(0, K // 16, row_scale, None)
        pltpu.sync_copy(rows_buf, s_sh.at[col_v.at[j]], add=True)
        return carry

    lax.fori_loop(0, NCH, chunk_body, None)
    plsc.subcore_barrier()
    for t in range(RPT // K):
        r0 = s * RPT + t * K
        pltpu.sync_copy(s_sh.at[pl.ds(r0, K)], rows_buf)
        pltpu.sync_copy(rows_buf, spart.at[pl.ds(c * P + r0, K)])


def _mm_body(x_ref, W_ref, u_ref, p0_ref, p1_ref, o_ref):
    W = W_ref[...]
    u0 = u_ref[...]
    v = jnp.dot(u0, W, preferred_element_type=jnp.float32)
    v = v / (jnp.sqrt(jnp.sum(v * v)) + 1e-12)
    u2 = lax.dot_general(v, W, (((1,), (1,)), ((), ())),
                         preferred_element_type=jnp.float32)
    u2 = u2 / (jnp.sqrt(jnp.sum(u2 * u2)) + 1e-12)
    Wv = lax.dot_general(W, v, (((1,), (1,)), ((), ())),
                         preferred_element_type=jnp.float32)
    sigma = jnp.dot(u2, Wv, preferred_element_type=jnp.float32)[0, 0]
    deg = p0_ref[...] + p1_ref[...] + 1.0
    di = lax.rsqrt(deg)[:, None]
    o_ref[...] = di * jnp.dot(x_ref[...], W,
                              preferred_element_type=jnp.float32) / sigma


def _cb_body(s0_ref, s1_ref, g_ref, p0_ref, p1_ref, b_ref, o_ref):
    deg = p0_ref[...] + p1_ref[...] + 1.0
    di = lax.rsqrt(deg)[:, None]
    o_ref[...] = di * (s0_ref[...] + s1_ref[...] + g_ref[...]) + b_ref[...]


_RB = 256


def kernel(x, edge_index, edge_wt, W, b, u):
    row = edge_index[0]
    col = edge_index[1]

    pad = EP_T - EW_T
    spread = (jnp.arange(NW * pad, dtype=jnp.int32) * 97) % N
    spread = spread.reshape(NW, pad)
    rowp = jnp.concatenate([row.reshape(NW, EW_T), spread], axis=1)
    colp = jnp.concatenate([col.reshape(NW, EW_T), spread], axis=1)
    ewp = jnp.concatenate(
        [edge_wt.reshape(NW, EW_T),
         jnp.full((NW, pad), -1e4, jnp.float32)], axis=1)
    rowp = rowp.reshape(NW, NCH, K)
    colp = colp.reshape(NW, NCH, K)
    ewp = ewp.reshape(NW, NCH, K)

    xp = jnp.pad(x, ((0, P - N), (0, 0)))

    degp = _deg_kernel(colp, ewp)

    g = pl.pallas_call(
        _mm_body,
        grid=(P // _RB,),
        in_specs=[
            pl.BlockSpec((_RB, FEAT), lambda i: (i, 0)),
            pl.BlockSpec((FEAT, HID), lambda i: (0, 0)),
            pl.BlockSpec((1, FEAT), lambda i: (0, 0)),
            pl.BlockSpec((_RB,), lambda i: (i,)),
            pl.BlockSpec((_RB,), lambda i: (i + P // _RB,)),
        ],
        out_specs=pl.BlockSpec((_RB, HID), lambda i: (i, 0)),
        out_shape=jax.ShapeDtypeStruct((P, HID), jnp.float32),
    )(xp, W, u.reshape(1, FEAT), degp, degp)

    spart = _msg_kernel(rowp, colp, ewp, g)

    out = pl.pallas_call(
        _cb_body,
        grid=(P // _RB,),
        in_specs=[
            pl.BlockSpec((_RB, HID), lambda i: (i, 0)),
            pl.BlockSpec((_RB, HID), lambda i: (i + P // _RB, 0)),
            pl.BlockSpec((_RB, HID), lambda i: (i, 0)),
            pl.BlockSpec((_RB,), lambda i: (i,)),
            pl.BlockSpec((_RB,), lambda i: (i + P // _RB,)),
            pl.BlockSpec((1, HID), lambda i: (0, 0)),
        ],
        out_specs=pl.BlockSpec((_RB, HID), lambda i: (i, 0)),
        out_shape=jax.ShapeDtypeStruct((P, HID), jnp.float32),
    )(spart, spart, g, degp, degp, b.reshape(1, HID))

    return out[:N]
